# Initial kernel scaffold; baseline (speedup 1.0000x reference)
import jax
import jax.numpy as jnp
from jax.experimental import pallas as pl


def kernel(x, edge_index, batch, W1, b1, W2, b2, W3, b3, W4, b4, conv5_w, conv5_b, conv6_w, conv6_b, fc1_w, fc1_b, fc2_w, fc2_b):
    raise NotImplementedError("write your pallas kernel here")


# speedup vs baseline: 1.0010x; 1.0010x over previous
import functools

import jax
import jax.numpy as jnp
from jax import lax
from jax.experimental import pallas as pl
from jax.experimental.pallas import tpu as pltpu

N_NODES = 10000
NUM_GRAPHS = 128
K_SORT = 30


def _gcn(x, src, dst, ew, W, b):
    n = x.shape[0]
    loop = jnp.arange(n, dtype=src.dtype)
    src_a = jnp.concatenate([src, loop])
    dst_a = jnp.concatenate([dst, loop])
    ew_a = jnp.concatenate([ew, jnp.ones((n,), jnp.float32)])
    xw = x @ W
    deg = jnp.zeros((n,), jnp.float32).at[dst_a].add(ew_a)
    dinv = jnp.where(deg > 0, 1.0 / jnp.sqrt(deg), 0.0)
    norm = dinv[src_a] * dinv[dst_a] * ew_a
    msg = xw[src_a] * norm[:, None]
    out = jnp.zeros((n, W.shape[1]), jnp.float32).at[dst_a].add(msg)
    return out + b


def _sort_pool(xc, batch, k, B):
    n, d = xc.shape
    keyv = batch.astype(jnp.float32) * 1e4 - xc[:, -1]
    order = jnp.argsort(keyv)
    xs = xc[order]
    bs = batch[order]
    counts = jnp.bincount(batch, length=B)
    starts = jnp.concatenate([jnp.zeros((1,), counts.dtype), jnp.cumsum(counts)[:-1]])
    rank = jnp.arange(n) - starts[bs]
    valid = rank < k
    slot = jnp.where(valid, rank, k)
    xs = jnp.where(valid[:, None], xs, 0.0)
    dense = jnp.zeros((B, k + 1, d), jnp.float32).at[bs, slot].set(xs)
    return dense[:, :k, :].reshape(B, k * d)


def _head_kernel(g_ref, w5_ref, b5_ref, w6_ref, b6_ref, fc1_ref, fc1b_ref,
                 fc2_ref, fc2b_ref, out_ref):
    g = g_ref[...]
    h1 = jnp.maximum(g @ w5_ref[...] + b5_ref[...], 0.0)
    h2 = jnp.maximum(h1[:, :240], h1[:, 240:])
    h3 = jnp.maximum(h2 @ w6_ref[...] + b6_ref[...], 0.0)
    h4 = jnp.maximum(h3 @ fc1_ref[...] + fc1b_ref[...], 0.0)
    logits = h4 @ fc2_ref[...] + fc2b_ref[...]
    m = jnp.max(logits, axis=-1, keepdims=True)
    s = logits - m
    lse = jnp.log(jnp.sum(jnp.exp(s), axis=-1, keepdims=True))
    out_ref[...] = s - lse


def _head(g, conv5_w, conv5_b, conv6_w, conv6_b, fc1_w, fc1_b, fc2_w, fc2_b):
    B = NUM_GRAPHS
    w5 = conv5_w[:, 0, :].T
    w5big = jnp.zeros((2910, 480), jnp.float32)
    for t in range(30):
        cb = (t // 2) * 16 + (240 if t % 2 else 0)
        w5big = w5big.at[t * 97:(t + 1) * 97, cb:cb + 16].set(w5)
    b5big = jnp.tile(conv5_b, 30)
    w6big = jnp.zeros((240, 352), jnp.float32)
    for t in range(11):
        for i in range(5):
            blk = conv6_w[:, :, i].T
            w6big = w6big.at[(t + i) * 16:(t + i + 1) * 16,
                             t * 32:(t + 1) * 32].set(blk)
    b6big = jnp.tile(conv6_b, 11)
    fc1p = fc1_w.reshape(32, 11, 128).transpose(1, 0, 2).reshape(352, 128)
    return pl.pallas_call(
        _head_kernel,
        out_shape=jax.ShapeDtypeStruct((B, 10), jnp.float32),
    )(g, w5big, b5big, w6big, b6big, fc1p, fc1_b, fc2_w, fc2_b)


def kernel(x, edge_index, batch, W1, b1, W2, b2, W3, b3, W4, b4,
           conv5_w, conv5_b, conv6_w, conv6_b,
           fc1_w, fc1_b, fc2_w, fc2_b):
    src = edge_index[0]
    dst = edge_index[1]
    ew = (src != dst).astype(jnp.float32)
    x1 = jnp.tanh(_gcn(x, src, dst, ew, W1, b1))
    x2 = jnp.tanh(_gcn(x1, src, dst, ew, W2, b2))
    x3 = jnp.tanh(_gcn(x2, src, dst, ew, W3, b3))
    x4 = jnp.tanh(_gcn(x3, src, dst, ew, W4, b4))
    xc = jnp.concatenate([x1, x2, x3, x4], axis=-1)
    g = _sort_pool(xc, batch, K_SORT, NUM_GRAPHS)
    return _head(g, conv5_w, conv5_b, conv6_w, conv6_b, fc1_w, fc1_b, fc2_w, fc2_b)


# speedup vs baseline: 13.4665x; 13.4536x over previous
import jax
import jax.numpy as jnp
from jax import lax
from jax.experimental import pallas as pl
from jax.experimental.pallas import tpu as pltpu
from jax.experimental.pallas import tpu_sc as plsc

N = 10000
E = 320000
ERP = 2560
B = 128
K = 30
NPD = 10240
NPS = 10016
DUMMY = N
NC, NS = 2, 16
NW = NC * NS
RW = 80
EGR = ERP // 4
ECH = 40
CHUNK = NPD // NS
TAB = 3968

_CP = pltpu.CompilerParams(needs_layout_passes=False)


def _mesh():
    return plsc.VectorSubcoreMesh(core_axis_name="c", subcore_axis_name="s",
                                  num_cores=NC, num_subcores=NS)


def _sc_scalar_body(with_table):
    def body(*args):
        if with_table:
            (src_ref, dst_ref, tab_ref, out_ref,
             accv, srcv, dstv, tabv, stack, aggv, totv) = args
        else:
            (src_ref, dst_ref, out_ref,
             accv, srcv, dstv, stack, aggv, totv) = args
        cid = lax.axis_index("c")
        sid = lax.axis_index("s")
        wid = cid * NS + sid
        zero16 = jnp.zeros((16,), jnp.float32)

        def zb(i, _):
            accv[pl.ds(i * 16, 16)] = zero16
            return 0
        lax.fori_loop(0, NPD // 16, zb, 0)
        if with_table:
            pltpu.sync_copy(tab_ref, tabv)
        start = wid * RW
        pltpu.sync_copy(src_ref.at[pl.ds(start, RW)], srcv)
        pltpu.sync_copy(dst_ref.at[pl.ds(start, RW)], dstv)
        ones = jnp.ones((16,), jnp.float32)

        def eb(r, _):
            for g in range(8):
                s16 = srcv[r, pl.ds(g * 16, 16)]
                d16 = dstv[r, pl.ds(g * 16, 16)]
                dp = jnp.where(s16 == d16, DUMMY, d16)
                if with_table:
                    v = plsc.load_gather(tabv, [s16])
                else:
                    v = ones
                plsc.addupdate_scatter(accv, [dp], v)
            return 0
        lax.fori_loop(0, RW, eb, 0)
        pltpu.sync_copy(accv, stack.at[sid])
        plsc.subcore_barrier()
        for t in range(NS):
            pltpu.sync_copy(stack.at[t, pl.ds(sid * CHUNK, CHUNK)], aggv.at[t])

        def rb(j, _):
            tot = aggv[0, pl.ds(j * 16, 16)]
            for t in range(1, NS):
                tot = tot + aggv[t, pl.ds(j * 16, 16)]
            totv[pl.ds(j * 16, 16)] = tot
            return 0
        lax.fori_loop(0, CHUNK // 16, rb, 0)
        pltpu.sync_copy(totv, out_ref.at[cid, pl.ds(sid * CHUNK, CHUNK)])
    return body


def _sc_scalar(src2, dst2, tab):
    scr = [pltpu.VMEM((NPD,), jnp.float32),
           pltpu.VMEM((RW, 128), jnp.int32),
           pltpu.VMEM((RW, 128), jnp.int32)]
    args = [src2, dst2]
    if tab is not None:
        scr.append(pltpu.VMEM((N,), jnp.float32))
        args.append(tab)
    scr += [pltpu.VMEM_SHARED((NS, NPD), jnp.float32),
            pltpu.VMEM((NS, CHUNK), jnp.float32),
            pltpu.VMEM((CHUNK,), jnp.float32)]
    return pl.kernel(_sc_scalar_body(tab is not None),
                     out_type=jax.ShapeDtypeStruct((NC, NPD), jnp.float32),
                     mesh=_mesh(), compiler_params=_CP,
                     scratch_types=scr)(*args)


def _sc_rows_body(y_ref, src_ref, dst_ref, out_ref, tabv, accv, srcc, dstc):
    cid = lax.axis_index("c")
    sid = lax.axis_index("s")
    wid = cid * NS + sid
    cg = wid % 8
    eg = wid // 8
    zero16 = jnp.zeros((16,), jnp.float32)
    for c in range(4):
        def zb(i, _, c=c):
            accv[c, pl.ds(i * 16, 16)] = zero16
            return 0
        lax.fori_loop(0, NPS // 16, zb, 0)
    pltpu.sync_copy(y_ref.at[cg], tabv)
    ccs = [jnp.full((16,), c, jnp.int32) for c in range(4)]
    for ch in range(EGR // ECH):
        base = eg * EGR + ch * ECH
        pltpu.sync_copy(src_ref.at[pl.ds(base, ECH)], srcc)
        pltpu.sync_copy(dst_ref.at[pl.ds(base, ECH)], dstc)

        def eb(r, _):
            for g in range(8):
                s16 = srcc[r, pl.ds(g * 16, 16)]
                d16 = dstc[r, pl.ds(g * 16, 16)]
                dp = jnp.where(s16 == d16, DUMMY, d16)
                for c in range(4):
                    v = plsc.load_gather(tabv, [ccs[c], s16])
                    plsc.addupdate_scatter(accv, [ccs[c], dp], v)
            return 0
        lax.fori_loop(0, ECH, eb, 0)
    pltpu.sync_copy(accv, out_ref.at[cid, sid])


def _sc_rows(yT, src2, dst2):
    return pl.kernel(_sc_rows_body,
                     out_type=jax.ShapeDtypeStruct((NC, NS, 4, NPS),
                                                   jnp.float32),
                     mesh=_mesh(), compiler_params=_CP,
                     scratch_types=[pltpu.VMEM((4, NPS), jnp.float32),
                                    pltpu.VMEM((4, NPS), jnp.float32),
                                    pltpu.VMEM((ECH, 128), jnp.int32),
                                    pltpu.VMEM((ECH, 128), jnp.int32)])(
        yT, src2, dst2)


def _table_body(pos_ref, xc_ref, g_ref, posv, tabv, idxv, rowb, tshared, sem):
    cid = lax.axis_index("c")
    sid = lax.axis_index("s")

    @pl.when(cid == 0)
    def _():
        @pl.when(sid == 0)
        def _():
            pltpu.sync_copy(pos_ref, posv)
            dum = jnp.full((16,), DUMMY, jnp.int32)

            def ib(i, _):
                tabv[pl.ds(i * 16, 16)] = dum
                return 0
            lax.fori_loop(0, TAB // 16, ib, 0)
            lane = lax.iota(jnp.int32, 16)

            def sb(k, _):
                p16 = posv[k // 8, pl.ds((k % 8) * 16, 16)]
                plsc.store_scatter(tabv, [p16], k * 16 + lane)
                return 0
            lax.fori_loop(0, NPD // 16, sb, 0)
            pltpu.sync_copy(tabv, tshared)
        plsc.subcore_barrier()
        for rep in range(2):
            j = sid * 2 + rep

            @pl.when(j < K)
            def _():
                pltpu.sync_copy(tshared.at[pl.ds(j * 128, 128)], idxv)
                pltpu.async_copy(xc_ref.at[idxv], rowb, sem).wait()
                pltpu.sync_copy(rowb, g_ref.at[pl.ds(j * 128, 128)])


def _table(pos2, xc):
    return pl.kernel(_table_body,
                     out_type=jax.ShapeDtypeStruct((K * B, 128), jnp.float32),
                     mesh=_mesh(), compiler_params=_CP,
                     scratch_types=[pltpu.VMEM((NPD // 128, 128), jnp.int32),
                                    pltpu.VMEM((TAB,), jnp.int32),
                                    pltpu.VMEM((128,), jnp.int32),
                                    pltpu.VMEM((128, 128), jnp.float32),
                                    pltpu.VMEM_SHARED((TAB,), jnp.int32),
                                    pltpu.SemaphoreType.DMA])(pos2, xc)


def _pre_body(xT_ref, w1T_ref, deg_ref, y1T_ref, dinvT_ref):
    d = deg_ref[...]
    cnt = d[0:1, :N] + d[1:2, :N]
    dinvT = 1.0 / jnp.sqrt(cnt + 1.0)
    xwT = jnp.dot(w1T_ref[...], xT_ref[...],
                  preferred_element_type=jnp.float32)
    y1T_ref[...] = jnp.concatenate(
        [xwT * dinvT, jnp.zeros((32, NPS - N), jnp.float32)], axis=1)
    dinvT_ref[...] = dinvT


def _pre(xT, W1T, deg):
    return pl.pallas_call(
        _pre_body,
        out_shape=(jax.ShapeDtypeStruct((32, NPS), jnp.float32),
                   jax.ShapeDtypeStruct((1, N), jnp.float32)))(xT, W1T, deg)


def _sum_parts(part):
    blocks = []
    for cg in range(8):
        blk = (part[0, cg] + part[0, cg + 8]
               + part[1, cg] + part[1, cg + 8])
        blocks.append(blk)
    return jnp.concatenate(blocks, axis=0)[:, :N]


def _upd_body(part_ref, yT_ref, dinvT_ref, bT_ref, wT_ref, xT_ref, ynT_ref):
    p = part_ref[...]
    sT = _sum_parts(p) + yT_ref[:, :N]
    dinvT = dinvT_ref[...]
    xT = jnp.tanh(dinvT * sT + bT_ref[...])
    xT_ref[...] = xT
    ynT = jnp.dot(wT_ref[...], xT,
                  preferred_element_type=jnp.float32) * dinvT
    dout = ynT.shape[0]
    ynT_ref[...] = jnp.concatenate(
        [ynT, jnp.zeros((dout, NPS - N), jnp.float32)], axis=1)


def _upd(part, yT, dinvT, b, WnT):
    dout = WnT.shape[0]
    return pl.pallas_call(
        _upd_body,
        out_shape=(jax.ShapeDtypeStruct((32, N), jnp.float32),
                   jax.ShapeDtypeStruct((dout, NPS), jnp.float32)))(
        part, yT, dinvT, b.reshape(-1, 1), WnT)


def _upd4_body(p4_ref, y4T_ref, dinvT_ref, b4_ref, x1T_ref, x2T_ref, x3T_ref,
               batch_ref, xcT_ref, keyrow_ref, starts_ref):
    p = p4_ref[...]
    s4T = p[0:1, :N] + p[1:2, :N] + y4T_ref[0:1, :N]
    x4T = jnp.tanh(dinvT_ref[...] * s4T + b4_ref[...])
    bT = batch_ref[...]
    keyrow = bT.astype(jnp.float32) * 1e4 - x4T
    keyrow_ref[...] = jnp.concatenate(
        [keyrow, jnp.full((1, NPD - N), jnp.inf, jnp.float32)], axis=1)
    xcT = jnp.concatenate([x1T_ref[...], x2T_ref[...], x3T_ref[...], x4T,
                           jnp.zeros((31, N), jnp.float32)], axis=0)
    xcT_ref[...] = jnp.concatenate(
        [xcT, jnp.zeros((128, NPS - N), jnp.float32)], axis=1)
    g_iota = lax.broadcasted_iota(jnp.int32, (B, 1), 0)
    starts_ref[...] = jnp.sum((bT < g_iota).astype(jnp.float32),
                              axis=1, keepdims=True).reshape(1, B)


def _upd4(p4, y4T, dinvT, b4, x1T, x2T, x3T, batchT):
    return pl.pallas_call(
        _upd4_body,
        out_shape=(jax.ShapeDtypeStruct((128, NPS), jnp.float32),
                   jax.ShapeDtypeStruct((1, NPD), jnp.float32),
                   jax.ShapeDtypeStruct((1, B), jnp.float32)))(
        p4, y4T, dinvT, b4.reshape(1, 1), x1T, x2T, x3T, batchT)


def _rank_body(ki_ref, kj_ref, bat_ref, starts_ref, pos_ref):
    i0 = pl.program_id(0) * 128
    ki = ki_ref[...]
    iidx = (i0 + lax.broadcasted_iota(jnp.int32, (128, 1), 0)
            ).astype(jnp.float32)
    rank = jnp.zeros((128, 1), jnp.float32)
    CJ = 512
    for c in range(NPD // CJ):
        kj = kj_ref[0:1, c * CJ:(c + 1) * CJ]
        jidx = (c * CJ + lax.broadcasted_iota(jnp.int32, (1, CJ), 1)
                ).astype(jnp.float32)
        lt = (kj < ki) | ((kj == ki) & (jidx < iidx))
        rank = rank + jnp.sum(jnp.where(lt, 1.0, 0.0), axis=1, keepdims=True)
    bat = bat_ref[...]
    oh = bat == lax.broadcasted_iota(jnp.int32, (1, B), 1)
    bstart = jnp.sum(jnp.where(oh, starts_ref[...], 0.0), axis=1,
                     keepdims=True)
    slot = rank - bstart
    pos = jnp.where(slot < float(K), bat.astype(jnp.float32) * K + slot,
                    float(K * B))
    pos_ref[...] = pos.astype(jnp.int32)


def _rank(keyp, keyrow, batchp, starts):
    return pl.pallas_call(
        _rank_body,
        grid=(NPD // 128,),
        in_specs=[pl.BlockSpec((128, 1), lambda i: (i, 0)),
                  pl.BlockSpec((1, NPD), lambda i: (0, 0)),
                  pl.BlockSpec((128, 1), lambda i: (i, 0)),
                  pl.BlockSpec((1, B), lambda i: (0, 0))],
        out_specs=pl.BlockSpec((128, 1), lambda i: (i, 0)),
        out_shape=jax.ShapeDtypeStruct((NPD, 1), jnp.int32))(
        keyp, keyrow, batchp, starts)


def _head_kernel(g_ref, w5_ref, b5_ref, w6_ref, b6_ref, fc1_ref, fc1b_ref,
                 fc2_ref, fc2b_ref, out_ref):
    g = g_ref[...]
    h1 = jnp.maximum(g @ w5_ref[...] + b5_ref[...], 0.0)
    h2 = jnp.maximum(h1[:, :240], h1[:, 240:])
    h3 = jnp.maximum(h2 @ w6_ref[...] + b6_ref[...], 0.0)
    h4 = jnp.maximum(h3 @ fc1_ref[...] + fc1b_ref[...], 0.0)
    logits = h4 @ fc2_ref[...] + fc2b_ref[...]
    m = jnp.max(logits, axis=-1, keepdims=True)
    s = logits - m
    lse = jnp.log(jnp.sum(jnp.exp(s), axis=-1, keepdims=True))
    out_ref[...] = s - lse


def _head(g, conv5_w, conv5_b, conv6_w, conv6_b, fc1_w, fc1_b, fc2_w, fc2_b):
    w5 = conv5_w[:, 0, :].T
    w5big = jnp.zeros((K * 128, 480), jnp.float32)
    for t in range(K):
        cb = (t // 2) * 16 + (240 if t % 2 else 0)
        w5big = w5big.at[t * 128:t * 128 + 97, cb:cb + 16].set(w5)
    b5big = jnp.tile(conv5_b, K)
    w6big = jnp.zeros((240, 352), jnp.float32)
    for t in range(11):
        for i in range(5):
            blk = conv6_w[:, :, i].T
            w6big = w6big.at[(t + i) * 16:(t + i + 1) * 16,
                             t * 32:(t + 1) * 32].set(blk)
    b6big = jnp.tile(conv6_b, 11)
    fc1p = fc1_w.reshape(32, 11, 128).transpose(1, 0, 2).reshape(352, 128)
    return pl.pallas_call(
        _head_kernel,
        out_shape=jax.ShapeDtypeStruct((B, 10), jnp.float32),
    )(g, w5big, b5big, w6big, b6big, fc1p, fc1_b, fc2_w, fc2_b)


def kernel(x, edge_index, batch, W1, b1, W2, b2, W3, b3, W4, b4,
           conv5_w, conv5_b, conv6_w, conv6_b,
           fc1_w, fc1_b, fc2_w, fc2_b):
    pad = jnp.zeros((ERP * 128 - E,), jnp.int32)
    src2 = jnp.concatenate([edge_index[0], pad]).reshape(ERP, 128)
    dst2 = jnp.concatenate([edge_index[1], pad]).reshape(ERP, 128)
    deg = _sc_scalar(src2, dst2, None)
    xT = x.T
    y1T, dinvT = _pre(xT, W1.T, deg)
    p1 = _sc_rows(y1T.reshape(8, 4, NPS), src2, dst2)
    x1T, y2T = _upd(p1, y1T, dinvT, b1, W2.T)
    p2 = _sc_rows(y2T.reshape(8, 4, NPS), src2, dst2)
    x2T, y3T = _upd(p2, y2T, dinvT, b2, W3.T)
    p3 = _sc_rows(y3T.reshape(8, 4, NPS), src2, dst2)
    x3T, y4T = _upd(p3, y3T, dinvT, b3, W4.T)
    p4 = _sc_scalar(src2, dst2, y4T[0, :N])
    xcT, keyrow, starts = _upd4(p4, y4T, dinvT, b4, x1T, x2T, x3T,
                                batch.reshape(1, N))
    xc = xcT.T
    keyp = keyrow.T
    batchp = jnp.concatenate(
        [batch.reshape(N, 1), jnp.zeros((NPD - N, 1), jnp.int32)], axis=0)
    pos = _rank(keyp, keyrow, batchp, starts)
    g = _table(pos.reshape(NPD // 128, 128), xc)
    return _head(g.reshape(B, K * 128), conv5_w, conv5_b, conv6_w, conv6_b,
                 fc1_w, fc1_b, fc2_w, fc2_b)


# speedup vs baseline: 18.0963x; 1.3438x over previous
import jax
import jax.numpy as jnp
from jax import lax
from jax.experimental import pallas as pl
from jax.experimental.pallas import tpu as pltpu
from jax.experimental.pallas import tpu_sc as plsc

N = 10000
E = 320000
ERP = 2560
B = 128
K = 30
NPD = 10240
NPS = 10016
DUMMY = N
NC, NS = 2, 16
NW = NC * NS
RW = 80
EGR = ERP // 4
ECH = 40
CHUNK = NPD // NS
TAB = 3968

_CP = pltpu.CompilerParams(needs_layout_passes=False)


def _mesh():
    return plsc.VectorSubcoreMesh(core_axis_name="c", subcore_axis_name="s",
                                  num_cores=NC, num_subcores=NS)


def _sc_scalar_body(with_table):
    def body(*args):
        if with_table:
            (src_ref, dst_ref, tab_ref, out_ref,
             accv, srcv, dstv, tabv, stack, aggv, totv) = args
        else:
            (src_ref, dst_ref, out_ref,
             accv, srcv, dstv, stack, aggv, totv) = args
        cid = lax.axis_index("c")
        sid = lax.axis_index("s")
        wid = cid * NS + sid
        zero16 = jnp.zeros((16,), jnp.float32)

        def zb(i, _):
            accv[pl.ds(i * 16, 16)] = zero16
            return 0
        lax.fori_loop(0, NPD // 16, zb, 0)
        if with_table:
            pltpu.sync_copy(tab_ref, tabv)
        start = wid * RW
        pltpu.sync_copy(src_ref.at[pl.ds(start, RW)], srcv)
        pltpu.sync_copy(dst_ref.at[pl.ds(start, RW)], dstv)
        ones = jnp.ones((16,), jnp.float32)

        def eb(r, _):
            for g in range(8):
                s16 = srcv[r, pl.ds(g * 16, 16)]
                d16 = dstv[r, pl.ds(g * 16, 16)]
                dp = jnp.where(s16 == d16, DUMMY, d16)
                if with_table:
                    v = plsc.load_gather(tabv, [s16])
                else:
                    v = ones
                plsc.addupdate_scatter(accv, [dp], v)
            return 0
        lax.fori_loop(0, RW, eb, 0)
        pltpu.sync_copy(accv, stack.at[sid])
        plsc.subcore_barrier()
        for t in range(NS):
            pltpu.sync_copy(stack.at[t, pl.ds(sid * CHUNK, CHUNK)], aggv.at[t])

        def rb(j, _):
            tot = aggv[0, pl.ds(j * 16, 16)]
            for t in range(1, NS):
                tot = tot + aggv[t, pl.ds(j * 16, 16)]
            totv[pl.ds(j * 16, 16)] = tot
            return 0
        lax.fori_loop(0, CHUNK // 16, rb, 0)
        pltpu.sync_copy(totv, out_ref.at[cid, pl.ds(sid * CHUNK, CHUNK)])
    return body


def _sc_scalar(src2, dst2, tab):
    scr = [pltpu.VMEM((NPD,), jnp.float32),
           pltpu.VMEM((RW, 128), jnp.int32),
           pltpu.VMEM((RW, 128), jnp.int32)]
    args = [src2, dst2]
    if tab is not None:
        scr.append(pltpu.VMEM((N,), jnp.float32))
        args.append(tab)
    scr += [pltpu.VMEM_SHARED((NS, NPD), jnp.float32),
            pltpu.VMEM((NS, CHUNK), jnp.float32),
            pltpu.VMEM((CHUNK,), jnp.float32)]
    return pl.kernel(_sc_scalar_body(tab is not None),
                     out_type=jax.ShapeDtypeStruct((NC, NPD), jnp.float32),
                     mesh=_mesh(), compiler_params=_CP,
                     scratch_types=scr)(*args)


def _sc_rows_body(y_ref, src_ref, dst_ref, out_ref, tabv, accv, srcc, dstc):
    cid = lax.axis_index("c")
    sid = lax.axis_index("s")
    wid = cid * NS + sid
    cg = wid % 8
    eg = wid // 8
    zero16 = jnp.zeros((16,), jnp.float32)

    def zb(i, _):
        for c in range(4):
            accv[c, pl.ds(i * 16, 16)] = zero16
        return 0
    lax.fori_loop(0, NPS // 16, zb, 0)
    pltpu.sync_copy(y_ref.at[cg], tabv)
    ccs = [jnp.full((16,), c, jnp.int32) for c in range(4)]
    for ch in range(EGR // ECH):
        base = eg * EGR + ch * ECH
        pltpu.sync_copy(src_ref.at[pl.ds(base, ECH)], srcc)
        pltpu.sync_copy(dst_ref.at[pl.ds(base, ECH)], dstc)

        def eb(r, _):
            for half in range(2):
                vs = []
                dps = []
                for g in range(half * 4, half * 4 + 4):
                    s16 = srcc[r, pl.ds(g * 16, 16)]
                    d16 = dstc[r, pl.ds(g * 16, 16)]
                    dps.append(jnp.where(s16 == d16, DUMMY, d16))
                    for c in range(4):
                        vs.append(plsc.load_gather(tabv, [ccs[c], s16]))
                for gi in range(4):
                    for c in range(4):
                        plsc.addupdate_scatter(accv, [ccs[c], dps[gi]],
                                               vs[gi * 4 + c])
            return 0
        lax.fori_loop(0, ECH, eb, 0)
    pltpu.sync_copy(accv, out_ref.at[cid, sid])


def _sc_rows(yT, src2, dst2):
    return pl.kernel(_sc_rows_body,
                     out_type=jax.ShapeDtypeStruct((NC, NS, 4, NPS),
                                                   jnp.float32),
                     mesh=_mesh(), compiler_params=_CP,
                     scratch_types=[pltpu.VMEM((4, NPS), jnp.float32),
                                    pltpu.VMEM((4, NPS), jnp.float32),
                                    pltpu.VMEM((ECH, 128), jnp.int32),
                                    pltpu.VMEM((ECH, 128), jnp.int32)])(
        yT, src2, dst2)


def _table_body(pos_ref, xc_ref, g_ref, posv, tabv, idxv, rowb, tshared, sem):
    cid = lax.axis_index("c")
    sid = lax.axis_index("s")

    @pl.when(cid == 0)
    def _():
        @pl.when(sid == 0)
        def _():
            pltpu.sync_copy(pos_ref, posv)
            dum = jnp.full((16,), DUMMY, jnp.int32)

            def ib(i, _):
                tabv[pl.ds(i * 16, 16)] = dum
                return 0
            lax.fori_loop(0, TAB // 16, ib, 0)
            lane = lax.iota(jnp.int32, 16)

            def sb(k, _):
                p16 = posv[k // 8, pl.ds((k % 8) * 16, 16)]
                plsc.store_scatter(tabv, [p16], k * 16 + lane)
                return 0
            lax.fori_loop(0, NPD // 16, sb, 0)
            pltpu.sync_copy(tabv, tshared)
        plsc.subcore_barrier()
        for rep in range(2):
            j = sid * 2 + rep

            @pl.when(j < K)
            def _():
                pltpu.sync_copy(tshared.at[pl.ds(j * 128, 128)], idxv)
                pltpu.async_copy(xc_ref.at[idxv], rowb, sem).wait()
                pltpu.sync_copy(rowb, g_ref.at[pl.ds(j * 128, 128)])


def _table(pos2, xc):
    return pl.kernel(_table_body,
                     out_type=jax.ShapeDtypeStruct((K * B, 128), jnp.float32),
                     mesh=_mesh(), compiler_params=_CP,
                     scratch_types=[pltpu.VMEM((NPD // 128, 128), jnp.int32),
                                    pltpu.VMEM((TAB,), jnp.int32),
                                    pltpu.VMEM((128,), jnp.int32),
                                    pltpu.VMEM((128, 128), jnp.float32),
                                    pltpu.VMEM_SHARED((TAB,), jnp.int32),
                                    pltpu.SemaphoreType.DMA])(pos2, xc)


def _pre_body(xT_ref, w1T_ref, deg_ref, y1T_ref, dinvT_ref):
    d = deg_ref[...]
    cnt = d[0:1, :N] + d[1:2, :N]
    dinvT = 1.0 / jnp.sqrt(cnt + 1.0)
    xwT = jnp.dot(w1T_ref[...], xT_ref[...],
                  preferred_element_type=jnp.float32)
    y1T_ref[...] = jnp.concatenate(
        [xwT * dinvT, jnp.zeros((32, NPS - N), jnp.float32)], axis=1)
    dinvT_ref[...] = dinvT


def _pre(xT, W1T, deg):
    return pl.pallas_call(
        _pre_body,
        out_shape=(jax.ShapeDtypeStruct((32, NPS), jnp.float32),
                   jax.ShapeDtypeStruct((1, N), jnp.float32)))(xT, W1T, deg)


def _sum_parts(part):
    blocks = []
    for cg in range(8):
        blk = (part[0, cg] + part[0, cg + 8]
               + part[1, cg] + part[1, cg + 8])
        blocks.append(blk)
    return jnp.concatenate(blocks, axis=0)[:, :N]


def _upd_body(part_ref, yT_ref, dinvT_ref, bT_ref, wT_ref, xT_ref, ynT_ref):
    p = part_ref[...]
    sT = _sum_parts(p) + yT_ref[:, :N]
    dinvT = dinvT_ref[...]
    xT = jnp.tanh(dinvT * sT + bT_ref[...])
    xT_ref[...] = xT
    ynT = jnp.dot(wT_ref[...], xT,
                  preferred_element_type=jnp.float32) * dinvT
    dout = ynT.shape[0]
    ynT_ref[...] = jnp.concatenate(
        [ynT, jnp.zeros((dout, NPS - N), jnp.float32)], axis=1)


def _upd(part, yT, dinvT, b, WnT):
    dout = WnT.shape[0]
    return pl.pallas_call(
        _upd_body,
        out_shape=(jax.ShapeDtypeStruct((32, N), jnp.float32),
                   jax.ShapeDtypeStruct((dout, NPS), jnp.float32)))(
        part, yT, dinvT, b.reshape(-1, 1), WnT)


def _upd4_body(p4_ref, y4T_ref, dinvT_ref, b4_ref, x1T_ref, x2T_ref, x3T_ref,
               batch_ref, xcT_ref, keyrow_ref, starts_ref):
    p = p4_ref[...]
    s4T = p[0:1, :N] + p[1:2, :N] + y4T_ref[0:1, :N]
    x4T = jnp.tanh(dinvT_ref[...] * s4T + b4_ref[...])
    bT = batch_ref[...]
    keyrow = bT.astype(jnp.float32) * 1e4 - x4T
    keyrow_ref[...] = jnp.concatenate(
        [keyrow, jnp.full((1, NPD - N), jnp.inf, jnp.float32)], axis=1)
    xcT = jnp.concatenate([x1T_ref[...], x2T_ref[...], x3T_ref[...], x4T,
                           jnp.zeros((31, N), jnp.float32)], axis=0)
    xcT_ref[...] = jnp.concatenate(
        [xcT, jnp.zeros((128, NPS - N), jnp.float32)], axis=1)
    g_iota = lax.broadcasted_iota(jnp.int32, (B, 1), 0)
    starts_ref[...] = jnp.sum((bT < g_iota).astype(jnp.float32),
                              axis=1, keepdims=True).reshape(1, B)


def _upd4(p4, y4T, dinvT, b4, x1T, x2T, x3T, batchT):
    return pl.pallas_call(
        _upd4_body,
        out_shape=(jax.ShapeDtypeStruct((128, NPS), jnp.float32),
                   jax.ShapeDtypeStruct((1, NPD), jnp.float32),
                   jax.ShapeDtypeStruct((1, B), jnp.float32)))(
        p4, y4T, dinvT, b4.reshape(1, 1), x1T, x2T, x3T, batchT)


def _rank_body(ki_ref, kj_ref, bat_ref, starts_ref, pos_ref):
    i0 = pl.program_id(0) * 128
    ki = ki_ref[...]
    iidx = (i0 + lax.broadcasted_iota(jnp.int32, (128, 1), 0)
            ).astype(jnp.float32)
    rank = jnp.zeros((128, 1), jnp.float32)
    CJ = 512
    for c in range(NPD // CJ):
        kj = kj_ref[0:1, c * CJ:(c + 1) * CJ]
        jidx = (c * CJ + lax.broadcasted_iota(jnp.int32, (1, CJ), 1)
                ).astype(jnp.float32)
        lt = (kj < ki) | ((kj == ki) & (jidx < iidx))
        rank = rank + jnp.sum(jnp.where(lt, 1.0, 0.0), axis=1, keepdims=True)
    bat = bat_ref[...]
    oh = bat == lax.broadcasted_iota(jnp.int32, (1, B), 1)
    bstart = jnp.sum(jnp.where(oh, starts_ref[...], 0.0), axis=1,
                     keepdims=True)
    slot = rank - bstart
    pos = jnp.where(slot < float(K), bat.astype(jnp.float32) * K + slot,
                    float(K * B))
    pos_ref[...] = pos.astype(jnp.int32)


def _rank(keyp, keyrow, batchp, starts):
    return pl.pallas_call(
        _rank_body,
        grid=(NPD // 128,),
        in_specs=[pl.BlockSpec((128, 1), lambda i: (i, 0)),
                  pl.BlockSpec((1, NPD), lambda i: (0, 0)),
                  pl.BlockSpec((128, 1), lambda i: (i, 0)),
                  pl.BlockSpec((1, B), lambda i: (0, 0))],
        out_specs=pl.BlockSpec((128, 1), lambda i: (i, 0)),
        out_shape=jax.ShapeDtypeStruct((NPD, 1), jnp.int32))(
        keyp, keyrow, batchp, starts)


def _head_kernel(g_ref, w5_ref, b5_ref, w6_ref, b6_ref, fc1_ref, fc1b_ref,
                 fc2_ref, fc2b_ref, out_ref):
    g = g_ref[...]
    h1 = jnp.maximum(g @ w5_ref[...] + b5_ref[...], 0.0)
    h2 = jnp.maximum(h1[:, :240], h1[:, 240:])
    h3 = jnp.maximum(h2 @ w6_ref[...] + b6_ref[...], 0.0)
    h4 = jnp.maximum(h3 @ fc1_ref[...] + fc1b_ref[...], 0.0)
    logits = h4 @ fc2_ref[...] + fc2b_ref[...]
    m = jnp.max(logits, axis=-1, keepdims=True)
    s = logits - m
    lse = jnp.log(jnp.sum(jnp.exp(s), axis=-1, keepdims=True))
    out_ref[...] = s - lse


def _head(g, conv5_w, conv5_b, conv6_w, conv6_b, fc1_w, fc1_b, fc2_w, fc2_b):
    w5 = conv5_w[:, 0, :].T
    w5big = jnp.zeros((K * 128, 480), jnp.float32)
    for t in range(K):
        cb = (t // 2) * 16 + (240 if t % 2 else 0)
        w5big = w5big.at[t * 128:t * 128 + 97, cb:cb + 16].set(w5)
    b5big = jnp.tile(conv5_b, K)
    w6big = jnp.zeros((240, 352), jnp.float32)
    for t in range(11):
        for i in range(5):
            blk = conv6_w[:, :, i].T
            w6big = w6big.at[(t + i) * 16:(t + i + 1) * 16,
                             t * 32:(t + 1) * 32].set(blk)
    b6big = jnp.tile(conv6_b, 11)
    fc1p = fc1_w.reshape(32, 11, 128).transpose(1, 0, 2).reshape(352, 128)
    return pl.pallas_call(
        _head_kernel,
        out_shape=jax.ShapeDtypeStruct((B, 10), jnp.float32),
    )(g, w5big, b5big, w6big, b6big, fc1p, fc1_b, fc2_w, fc2_b)


def kernel(x, edge_index, batch, W1, b1, W2, b2, W3, b3, W4, b4,
           conv5_w, conv5_b, conv6_w, conv6_b,
           fc1_w, fc1_b, fc2_w, fc2_b):
    pad = jnp.zeros((ERP * 128 - E,), jnp.int32)
    src2 = jnp.concatenate([edge_index[0], pad]).reshape(ERP, 128)
    dst2 = jnp.concatenate([edge_index[1], pad]).reshape(ERP, 128)
    deg = _sc_scalar(src2, dst2, None)
    xT = x.T
    y1T, dinvT = _pre(xT, W1.T, deg)
    p1 = _sc_rows(y1T.reshape(8, 4, NPS), src2, dst2)
    x1T, y2T = _upd(p1, y1T, dinvT, b1, W2.T)
    p2 = _sc_rows(y2T.reshape(8, 4, NPS), src2, dst2)
    x2T, y3T = _upd(p2, y2T, dinvT, b2, W3.T)
    p3 = _sc_rows(y3T.reshape(8, 4, NPS), src2, dst2)
    x3T, y4T = _upd(p3, y3T, dinvT, b3, W4.T)
    p4 = _sc_scalar(src2, dst2, y4T[0, :N])
    xcT, keyrow, starts = _upd4(p4, y4T, dinvT, b4, x1T, x2T, x3T,
                                batch.reshape(1, N))
    xc = xcT.T
    keyp = keyrow.T
    batchp = jnp.concatenate(
        [batch.reshape(N, 1), jnp.zeros((NPD - N, 1), jnp.int32)], axis=0)
    pos = _rank(keyp, keyrow, batchp, starts)
    g = _table(pos.reshape(NPD // 128, 128), xc)
    return _head(g.reshape(B, K * 128), conv5_w, conv5_b, conv6_w, conv6_b,
                 fc1_w, fc1_b, fc2_w, fc2_b)


# speedup vs baseline: 18.2001x; 1.0057x over previous
import jax
import jax.numpy as jnp
from jax import lax
from jax.experimental import pallas as pl
from jax.experimental.pallas import tpu as pltpu
from jax.experimental.pallas import tpu_sc as plsc

N = 10000
E = 320000
ERP = 2560
B = 128
K = 30
NPD = 10240
NPS = 10016
DUMMY = N
NC, NS = 2, 16
NW = NC * NS
RW = 80
EGR = ERP // 4
ECH = 40
CHUNK = NPD // NS
TAB = 3968

_CP = pltpu.CompilerParams(needs_layout_passes=False)


def _mesh():
    return plsc.VectorSubcoreMesh(core_axis_name="c", subcore_axis_name="s",
                                  num_cores=NC, num_subcores=NS)


def _sc_scalar_body(with_table):
    def body(*args):
        if with_table:
            (src_ref, dst_ref, tab_ref, out_ref,
             accv, srcv, dstv, tabv, stack, aggv, totv) = args
        else:
            (src_ref, dst_ref, out_ref,
             accv, srcv, dstv, stack, aggv, totv) = args
        cid = lax.axis_index("c")
        sid = lax.axis_index("s")
        wid = cid * NS + sid
        zero16 = jnp.zeros((16,), jnp.float32)

        def zb(i, _):
            accv[pl.ds(i * 16, 16)] = zero16
            return 0
        lax.fori_loop(0, NPD // 16, zb, 0)
        if with_table:
            pltpu.sync_copy(tab_ref, tabv)
        start = wid * RW
        pltpu.sync_copy(src_ref.at[pl.ds(start, RW)], srcv)
        pltpu.sync_copy(dst_ref.at[pl.ds(start, RW)], dstv)
        ones = jnp.ones((16,), jnp.float32)

        def eb(r, _):
            for g in range(8):
                s16 = srcv[r, pl.ds(g * 16, 16)]
                d16 = dstv[r, pl.ds(g * 16, 16)]
                dp = jnp.where(s16 == d16, DUMMY, d16)
                if with_table:
                    v = plsc.load_gather(tabv, [s16])
                else:
                    v = ones
                plsc.addupdate_scatter(accv, [dp], v)
            return 0
        lax.fori_loop(0, RW, eb, 0)
        pltpu.sync_copy(accv, stack.at[sid])
        plsc.subcore_barrier()
        for t in range(NS):
            pltpu.sync_copy(stack.at[t, pl.ds(sid * CHUNK, CHUNK)], aggv.at[t])

        def rb(j, _):
            tot = aggv[0, pl.ds(j * 16, 16)]
            for t in range(1, NS):
                tot = tot + aggv[t, pl.ds(j * 16, 16)]
            totv[pl.ds(j * 16, 16)] = tot
            return 0
        lax.fori_loop(0, CHUNK // 16, rb, 0)
        pltpu.sync_copy(totv, out_ref.at[cid, pl.ds(sid * CHUNK, CHUNK)])
    return body


def _sc_scalar(src2, dst2, tab):
    scr = [pltpu.VMEM((NPD,), jnp.float32),
           pltpu.VMEM((RW, 128), jnp.int32),
           pltpu.VMEM((RW, 128), jnp.int32)]
    args = [src2, dst2]
    if tab is not None:
        scr.append(pltpu.VMEM((N,), jnp.float32))
        args.append(tab)
    scr += [pltpu.VMEM_SHARED((NS, NPD), jnp.float32),
            pltpu.VMEM((NS, CHUNK), jnp.float32),
            pltpu.VMEM((CHUNK,), jnp.float32)]
    return pl.kernel(_sc_scalar_body(tab is not None),
                     out_type=jax.ShapeDtypeStruct((NC, NPD), jnp.float32),
                     mesh=_mesh(), compiler_params=_CP,
                     scratch_types=scr)(*args)


def _sc_rows_body(y_ref, src_ref, dst_ref, out_ref, tabv, accv, srcc, dstc):
    cid = lax.axis_index("c")
    sid = lax.axis_index("s")
    wid = cid * NS + sid
    cg = wid % 8
    eg = wid // 8
    zero16 = jnp.zeros((16,), jnp.float32)

    def zb(i, _):
        for c in range(4):
            accv[c, pl.ds(i * 16, 16)] = zero16
        return 0
    lax.fori_loop(0, NPS // 16, zb, 0)
    pltpu.sync_copy(y_ref.at[cg], tabv)
    ccs = [jnp.full((16,), c, jnp.int32) for c in range(4)]
    for ch in range(EGR // ECH):
        base = eg * EGR + ch * ECH
        pltpu.sync_copy(src_ref.at[pl.ds(base, ECH)], srcc)
        pltpu.sync_copy(dst_ref.at[pl.ds(base, ECH)], dstc)

        def eb(r, _):
            vs = []
            dps = []
            for g in range(8):
                s16 = srcc[r, pl.ds(g * 16, 16)]
                d16 = dstc[r, pl.ds(g * 16, 16)]
                dps.append(jnp.where(s16 == d16, DUMMY, d16))
                for c in range(4):
                    vs.append(plsc.load_gather(tabv, [ccs[c], s16]))
            for gi in range(8):
                for c in range(4):
                    plsc.addupdate_scatter(accv, [ccs[c], dps[gi]],
                                           vs[gi * 4 + c])
            return 0
        lax.fori_loop(0, ECH, eb, 0)
    pltpu.sync_copy(accv, out_ref.at[cid, sid])


def _sc_rows(yT, src2, dst2):
    return pl.kernel(_sc_rows_body,
                     out_type=jax.ShapeDtypeStruct((NC, NS, 4, NPS),
                                                   jnp.float32),
                     mesh=_mesh(), compiler_params=_CP,
                     scratch_types=[pltpu.VMEM((4, NPS), jnp.float32),
                                    pltpu.VMEM((4, NPS), jnp.float32),
                                    pltpu.VMEM((ECH, 128), jnp.int32),
                                    pltpu.VMEM((ECH, 128), jnp.int32)])(
        yT, src2, dst2)


def _table_body(pos_ref, xc_ref, g_ref, posv, tabv, idxv, rowb, tshared, sem):
    cid = lax.axis_index("c")
    sid = lax.axis_index("s")

    @pl.when(cid == 0)
    def _():
        @pl.when(sid == 0)
        def _():
            pltpu.sync_copy(pos_ref, posv)
            dum = jnp.full((16,), DUMMY, jnp.int32)

            def ib(i, _):
                tabv[pl.ds(i * 16, 16)] = dum
                return 0
            lax.fori_loop(0, TAB // 16, ib, 0)
            lane = lax.iota(jnp.int32, 16)

            def sb(k, _):
                p16 = posv[k // 8, pl.ds((k % 8) * 16, 16)]
                plsc.store_scatter(tabv, [p16], k * 16 + lane)
                return 0
            lax.fori_loop(0, NPD // 16, sb, 0)
            pltpu.sync_copy(tabv, tshared)
        plsc.subcore_barrier()
        for rep in range(2):
            j = sid * 2 + rep

            @pl.when(j < K)
            def _():
                pltpu.sync_copy(tshared.at[pl.ds(j * 128, 128)], idxv)
                pltpu.async_copy(xc_ref.at[idxv], rowb, sem).wait()
                pltpu.sync_copy(rowb, g_ref.at[pl.ds(j * 128, 128)])


def _table(pos2, xc):
    return pl.kernel(_table_body,
                     out_type=jax.ShapeDtypeStruct((K * B, 128), jnp.float32),
                     mesh=_mesh(), compiler_params=_CP,
                     scratch_types=[pltpu.VMEM((NPD // 128, 128), jnp.int32),
                                    pltpu.VMEM((TAB,), jnp.int32),
                                    pltpu.VMEM((128,), jnp.int32),
                                    pltpu.VMEM((128, 128), jnp.float32),
                                    pltpu.VMEM_SHARED((TAB,), jnp.int32),
                                    pltpu.SemaphoreType.DMA])(pos2, xc)


def _pre_body(xT_ref, w1T_ref, deg_ref, y1T_ref, dinvT_ref):
    d = deg_ref[...]
    cnt = d[0:1, :N] + d[1:2, :N]
    dinvT = 1.0 / jnp.sqrt(cnt + 1.0)
    xwT = jnp.dot(w1T_ref[...], xT_ref[...],
                  preferred_element_type=jnp.float32)
    y1T_ref[...] = jnp.concatenate(
        [xwT * dinvT, jnp.zeros((32, NPS - N), jnp.float32)], axis=1)
    dinvT_ref[...] = dinvT


def _pre(xT, W1T, deg):
    return pl.pallas_call(
        _pre_body,
        out_shape=(jax.ShapeDtypeStruct((32, NPS), jnp.float32),
                   jax.ShapeDtypeStruct((1, N), jnp.float32)))(xT, W1T, deg)


def _sum_parts(part):
    blocks = []
    for cg in range(8):
        blk = (part[0, cg] + part[0, cg + 8]
               + part[1, cg] + part[1, cg + 8])
        blocks.append(blk)
    return jnp.concatenate(blocks, axis=0)[:, :N]


def _upd_body(part_ref, yT_ref, dinvT_ref, bT_ref, wT_ref, xT_ref, ynT_ref):
    p = part_ref[...]
    sT = _sum_parts(p) + yT_ref[:, :N]
    dinvT = dinvT_ref[...]
    xT = jnp.tanh(dinvT * sT + bT_ref[...])
    xT_ref[...] = xT
    ynT = jnp.dot(wT_ref[...], xT,
                  preferred_element_type=jnp.float32) * dinvT
    dout = ynT.shape[0]
    ynT_ref[...] = jnp.concatenate(
        [ynT, jnp.zeros((dout, NPS - N), jnp.float32)], axis=1)


def _upd(part, yT, dinvT, b, WnT):
    dout = WnT.shape[0]
    return pl.pallas_call(
        _upd_body,
        out_shape=(jax.ShapeDtypeStruct((32, N), jnp.float32),
                   jax.ShapeDtypeStruct((dout, NPS), jnp.float32)))(
        part, yT, dinvT, b.reshape(-1, 1), WnT)


def _upd4_body(p4_ref, y4T_ref, dinvT_ref, b4_ref, x1T_ref, x2T_ref, x3T_ref,
               batch_ref, xcT_ref, keyrow_ref, starts_ref):
    p = p4_ref[...]
    s4T = p[0:1, :N] + p[1:2, :N] + y4T_ref[0:1, :N]
    x4T = jnp.tanh(dinvT_ref[...] * s4T + b4_ref[...])
    bT = batch_ref[...]
    keyrow = bT.astype(jnp.float32) * 1e4 - x4T
    keyrow_ref[...] = jnp.concatenate(
        [keyrow, jnp.full((1, NPD - N), jnp.inf, jnp.float32)], axis=1)
    xcT = jnp.concatenate([x1T_ref[...], x2T_ref[...], x3T_ref[...], x4T,
                           jnp.zeros((31, N), jnp.float32)], axis=0)
    xcT_ref[...] = jnp.concatenate(
        [xcT, jnp.zeros((128, NPS - N), jnp.float32)], axis=1)
    g_iota = lax.broadcasted_iota(jnp.int32, (B, 1), 0)
    starts_ref[...] = jnp.sum((bT < g_iota).astype(jnp.float32),
                              axis=1, keepdims=True).reshape(1, B)


def _upd4(p4, y4T, dinvT, b4, x1T, x2T, x3T, batchT):
    return pl.pallas_call(
        _upd4_body,
        out_shape=(jax.ShapeDtypeStruct((128, NPS), jnp.float32),
                   jax.ShapeDtypeStruct((1, NPD), jnp.float32),
                   jax.ShapeDtypeStruct((1, B), jnp.float32)))(
        p4, y4T, dinvT, b4.reshape(1, 1), x1T, x2T, x3T, batchT)


def _rank_body(ki_ref, kj_ref, bat_ref, starts_ref, pos_ref):
    i0 = pl.program_id(0) * 128
    ki = ki_ref[...]
    iidx = (i0 + lax.broadcasted_iota(jnp.int32, (128, 1), 0)
            ).astype(jnp.float32)
    rank = jnp.zeros((128, 1), jnp.float32)
    CJ = 512
    for c in range(NPD // CJ):
        kj = kj_ref[0:1, c * CJ:(c + 1) * CJ]
        jidx = (c * CJ + lax.broadcasted_iota(jnp.int32, (1, CJ), 1)
                ).astype(jnp.float32)
        lt = (kj < ki) | ((kj == ki) & (jidx < iidx))
        rank = rank + jnp.sum(jnp.where(lt, 1.0, 0.0), axis=1, keepdims=True)
    bat = bat_ref[...]
    oh = bat == lax.broadcasted_iota(jnp.int32, (1, B), 1)
    bstart = jnp.sum(jnp.where(oh, starts_ref[...], 0.0), axis=1,
                     keepdims=True)
    slot = rank - bstart
    pos = jnp.where(slot < float(K), bat.astype(jnp.float32) * K + slot,
                    float(K * B))
    pos_ref[...] = pos.astype(jnp.int32)


def _rank(keyp, keyrow, batchp, starts):
    return pl.pallas_call(
        _rank_body,
        grid=(NPD // 128,),
        in_specs=[pl.BlockSpec((128, 1), lambda i: (i, 0)),
                  pl.BlockSpec((1, NPD), lambda i: (0, 0)),
                  pl.BlockSpec((128, 1), lambda i: (i, 0)),
                  pl.BlockSpec((1, B), lambda i: (0, 0))],
        out_specs=pl.BlockSpec((128, 1), lambda i: (i, 0)),
        out_shape=jax.ShapeDtypeStruct((NPD, 1), jnp.int32))(
        keyp, keyrow, batchp, starts)


def _head_kernel(g_ref, w5_ref, b5_ref, w6_ref, b6_ref, fc1_ref, fc1b_ref,
                 fc2_ref, fc2b_ref, out_ref):
    g = g_ref[...]
    h1 = jnp.maximum(g @ w5_ref[...] + b5_ref[...], 0.0)
    h2 = jnp.maximum(h1[:, :240], h1[:, 240:])
    h3 = jnp.maximum(h2 @ w6_ref[...] + b6_ref[...], 0.0)
    h4 = jnp.maximum(h3 @ fc1_ref[...] + fc1b_ref[...], 0.0)
    logits = h4 @ fc2_ref[...] + fc2b_ref[...]
    m = jnp.max(logits, axis=-1, keepdims=True)
    s = logits - m
    lse = jnp.log(jnp.sum(jnp.exp(s), axis=-1, keepdims=True))
    out_ref[...] = s - lse


def _head(g, conv5_w, conv5_b, conv6_w, conv6_b, fc1_w, fc1_b, fc2_w, fc2_b):
    w5 = conv5_w[:, 0, :].T
    w5big = jnp.zeros((K * 128, 480), jnp.float32)
    for t in range(K):
        cb = (t // 2) * 16 + (240 if t % 2 else 0)
        w5big = w5big.at[t * 128:t * 128 + 97, cb:cb + 16].set(w5)
    b5big = jnp.tile(conv5_b, K)
    w6big = jnp.zeros((240, 352), jnp.float32)
    for t in range(11):
        for i in range(5):
            blk = conv6_w[:, :, i].T
            w6big = w6big.at[(t + i) * 16:(t + i + 1) * 16,
                             t * 32:(t + 1) * 32].set(blk)
    b6big = jnp.tile(conv6_b, 11)
    fc1p = fc1_w.reshape(32, 11, 128).transpose(1, 0, 2).reshape(352, 128)
    return pl.pallas_call(
        _head_kernel,
        out_shape=jax.ShapeDtypeStruct((B, 10), jnp.float32),
    )(g, w5big, b5big, w6big, b6big, fc1p, fc1_b, fc2_w, fc2_b)


def kernel(x, edge_index, batch, W1, b1, W2, b2, W3, b3, W4, b4,
           conv5_w, conv5_b, conv6_w, conv6_b,
           fc1_w, fc1_b, fc2_w, fc2_b):
    pad = jnp.zeros((ERP * 128 - E,), jnp.int32)
    src2 = jnp.concatenate([edge_index[0], pad]).reshape(ERP, 128)
    dst2 = jnp.concatenate([edge_index[1], pad]).reshape(ERP, 128)
    deg = _sc_scalar(src2, dst2, None)
    xT = x.T
    y1T, dinvT = _pre(xT, W1.T, deg)
    p1 = _sc_rows(y1T.reshape(8, 4, NPS), src2, dst2)
    x1T, y2T = _upd(p1, y1T, dinvT, b1, W2.T)
    p2 = _sc_rows(y2T.reshape(8, 4, NPS), src2, dst2)
    x2T, y3T = _upd(p2, y2T, dinvT, b2, W3.T)
    p3 = _sc_rows(y3T.reshape(8, 4, NPS), src2, dst2)
    x3T, y4T = _upd(p3, y3T, dinvT, b3, W4.T)
    p4 = _sc_scalar(src2, dst2, y4T[0, :N])
    xcT, keyrow, starts = _upd4(p4, y4T, dinvT, b4, x1T, x2T, x3T,
                                batch.reshape(1, N))
    xc = xcT.T
    keyp = keyrow.T
    batchp = jnp.concatenate(
        [batch.reshape(N, 1), jnp.zeros((NPD - N, 1), jnp.int32)], axis=0)
    pos = _rank(keyp, keyrow, batchp, starts)
    g = _table(pos.reshape(NPD // 128, 128), xc)
    return _head(g.reshape(B, K * 128), conv5_w, conv5_b, conv6_w, conv6_b,
                 fc1_w, fc1_b, fc2_w, fc2_b)


# speedup vs baseline: 19.7965x; 1.0877x over previous
import jax
import jax.numpy as jnp
from jax import lax
from jax.experimental import pallas as pl
from jax.experimental.pallas import tpu as pltpu
from jax.experimental.pallas import tpu_sc as plsc

N = 10000
E = 320000
ERP = 2560
B = 128
K = 30
NPD = 10240
NPS = 10016
DUMMY = N
NC, NS = 2, 16
NW = NC * NS
RW = 80
EGR = ERP // 4
ECH = 40
CHUNK = NPD // NS
TAB = 3968

_CP = pltpu.CompilerParams(needs_layout_passes=False)


def _mesh():
    return plsc.VectorSubcoreMesh(core_axis_name="c", subcore_axis_name="s",
                                  num_cores=NC, num_subcores=NS)


def _sc_scalar_body(with_table):
    def body(*args):
        if with_table:
            (src_ref, dst_ref, tab_ref, out_ref,
             accv, srcv, dstv, tabv, stack, aggv, totv) = args
        else:
            (src_ref, dst_ref, out_ref,
             accv, srcv, dstv, stack, aggv, totv) = args
        cid = lax.axis_index("c")
        sid = lax.axis_index("s")
        wid = cid * NS + sid
        zero16 = jnp.zeros((16,), jnp.float32)

        def zb(i, _):
            accv[pl.ds(i * 16, 16)] = zero16
            return 0
        lax.fori_loop(0, NPD // 16, zb, 0)
        if with_table:
            pltpu.sync_copy(tab_ref, tabv)
        start = wid * RW
        pltpu.sync_copy(src_ref.at[pl.ds(start, RW)], srcv)
        pltpu.sync_copy(dst_ref.at[pl.ds(start, RW)], dstv)
        ones = jnp.ones((16,), jnp.float32)

        def eb(r, _):
            for g in range(8):
                s16 = srcv[r, pl.ds(g * 16, 16)]
                d16 = dstv[r, pl.ds(g * 16, 16)]
                dp = jnp.where(s16 == d16, DUMMY, d16)
                if with_table:
                    v = plsc.load_gather(tabv, [s16])
                else:
                    v = ones
                plsc.addupdate_scatter(accv, [dp], v)
            return 0
        lax.fori_loop(0, RW, eb, 0)
        pltpu.sync_copy(accv, stack.at[sid])
        plsc.subcore_barrier()
        for t in range(NS):
            pltpu.sync_copy(stack.at[t, pl.ds(sid * CHUNK, CHUNK)], aggv.at[t])

        def rb(j, _):
            tot = aggv[0, pl.ds(j * 16, 16)]
            for t in range(1, NS):
                tot = tot + aggv[t, pl.ds(j * 16, 16)]
            totv[pl.ds(j * 16, 16)] = tot
            return 0
        lax.fori_loop(0, CHUNK // 16, rb, 0)
        pltpu.sync_copy(totv, out_ref.at[cid, pl.ds(sid * CHUNK, CHUNK)])
    return body


def _sc_scalar(src2, dst2, tab):
    scr = [pltpu.VMEM((NPD,), jnp.float32),
           pltpu.VMEM((RW, 128), jnp.int32),
           pltpu.VMEM((RW, 128), jnp.int32)]
    args = [src2, dst2]
    if tab is not None:
        scr.append(pltpu.VMEM((N,), jnp.float32))
        args.append(tab)
    scr += [pltpu.VMEM_SHARED((NS, NPD), jnp.float32),
            pltpu.VMEM((NS, CHUNK), jnp.float32),
            pltpu.VMEM((CHUNK,), jnp.float32)]
    return pl.kernel(_sc_scalar_body(tab is not None),
                     out_type=jax.ShapeDtypeStruct((NC, NPD), jnp.float32),
                     mesh=_mesh(), compiler_params=_CP,
                     scratch_types=scr)(*args)


def _sc_rows_body(y_ref, src_ref, dst_ref, out_ref, tabv, accv, srcc, dstc):
    cid = lax.axis_index("c")
    sid = lax.axis_index("s")
    wid = cid * NS + sid
    cg = wid % 8
    eg = wid // 8
    zero16 = jnp.zeros((16,), jnp.float32)

    def zb(i, _):
        for c in range(4):
            accv[c, pl.ds(i * 16, 16)] = zero16
        return 0
    lax.fori_loop(0, NPS // 16, zb, 0)
    pltpu.sync_copy(y_ref.at[cg], tabv)
    ccs = [jnp.full((16,), c, jnp.int32) for c in range(4)]
    for ch in range(EGR // ECH):
        base = eg * EGR + ch * ECH
        pltpu.sync_copy(src_ref.at[pl.ds(base, ECH)], srcc)
        pltpu.sync_copy(dst_ref.at[pl.ds(base, ECH)], dstc)

        def eb(r, _):
            vs = []
            dps = []
            for g in range(8):
                s16 = srcc[r, pl.ds(g * 16, 16)]
                d16 = dstc[r, pl.ds(g * 16, 16)]
                dps.append(jnp.where(s16 == d16, DUMMY, d16))
                for c in range(4):
                    vs.append(plsc.load_gather(tabv, [ccs[c], s16]))
            for gi in range(8):
                for c in range(4):
                    plsc.addupdate_scatter(accv, [ccs[c], dps[gi]],
                                           vs[gi * 4 + c])
            return 0
        lax.fori_loop(0, ECH, eb, 0)
    pltpu.sync_copy(accv, out_ref.at[cid, sid])


def _sc_rows(yT, src2, dst2):
    return pl.kernel(_sc_rows_body,
                     out_type=jax.ShapeDtypeStruct((NC, NS, 4, NPS),
                                                   jnp.float32),
                     mesh=_mesh(), compiler_params=_CP,
                     scratch_types=[pltpu.VMEM((4, NPS), jnp.float32),
                                    pltpu.VMEM((4, NPS), jnp.float32),
                                    pltpu.VMEM((ECH, 128), jnp.int32),
                                    pltpu.VMEM((ECH, 128), jnp.int32)])(
        yT, src2, dst2)


def _table_body(pos_ref, xc_ref, g_ref, posv, tabv, idxv, rowb, tshared, sem):
    cid = lax.axis_index("c")
    sid = lax.axis_index("s")

    @pl.when(cid == 0)
    def _():
        @pl.when(sid == 0)
        def _():
            pltpu.sync_copy(pos_ref, posv)
            dum = jnp.full((16,), DUMMY, jnp.int32)

            def ib(i, _):
                tabv[pl.ds(i * 16, 16)] = dum
                return 0
            lax.fori_loop(0, TAB // 16, ib, 0)
            lane = lax.iota(jnp.int32, 16)

            def sb(k, _):
                p16 = posv[k // 8, pl.ds((k % 8) * 16, 16)]
                ids = jnp.minimum(k * 16 + lane, DUMMY)
                plsc.store_scatter(tabv, [p16], ids)
                return 0
            lax.fori_loop(0, NPD // 16, sb, 0)
            pltpu.sync_copy(tabv, tshared)
        plsc.subcore_barrier()
        for rep in range(2):
            j = sid * 2 + rep

            @pl.when(j < K)
            def _():
                pltpu.sync_copy(tshared.at[pl.ds(j * 128, 128)], idxv)
                pltpu.async_copy(xc_ref.at[idxv], rowb, sem).wait()
                pltpu.sync_copy(rowb, g_ref.at[pl.ds(j * 128, 128)])


def _table(pos2, xc):
    return pl.kernel(_table_body,
                     out_type=jax.ShapeDtypeStruct((K * B, 128), jnp.float32),
                     mesh=_mesh(), compiler_params=_CP,
                     scratch_types=[pltpu.VMEM((NPD // 128, 128), jnp.int32),
                                    pltpu.VMEM((TAB,), jnp.int32),
                                    pltpu.VMEM((128,), jnp.int32),
                                    pltpu.VMEM((128, 128), jnp.float32),
                                    pltpu.VMEM_SHARED((TAB,), jnp.int32),
                                    pltpu.SemaphoreType.DMA])(pos2, xc)


def _pre_body(xT_ref, w1T_ref, deg_ref, y1T_ref, dinvT_ref):
    d = deg_ref[...]
    cnt = d[0:1, :N] + d[1:2, :N]
    dinvT = 1.0 / jnp.sqrt(cnt + 1.0)
    xwT = jnp.dot(w1T_ref[...], xT_ref[...],
                  preferred_element_type=jnp.float32)
    y1T_ref[...] = jnp.concatenate(
        [xwT * dinvT, jnp.zeros((32, NPS - N), jnp.float32)], axis=1)
    dinvT_ref[...] = dinvT


def _pre(xT, W1T, deg):
    return pl.pallas_call(
        _pre_body,
        out_shape=(jax.ShapeDtypeStruct((32, NPS), jnp.float32),
                   jax.ShapeDtypeStruct((1, N), jnp.float32)))(xT, W1T, deg)


def _sum_parts(part):
    blocks = []
    for cg in range(8):
        blk = (part[0, cg] + part[0, cg + 8]
               + part[1, cg] + part[1, cg + 8])
        blocks.append(blk)
    return jnp.concatenate(blocks, axis=0)[:, :N]


def _upd_body(part_ref, yT_ref, dinvT_ref, bT_ref, wT_ref, xT_ref, ynT_ref):
    p = part_ref[...]
    sT = _sum_parts(p) + yT_ref[:, :N]
    dinvT = dinvT_ref[...]
    xT = jnp.tanh(dinvT * sT + bT_ref[...])
    xT_ref[...] = xT
    ynT = jnp.dot(wT_ref[...], xT,
                  preferred_element_type=jnp.float32) * dinvT
    dout = ynT.shape[0]
    ynT_ref[...] = jnp.concatenate(
        [ynT, jnp.zeros((dout, NPS - N), jnp.float32)], axis=1)


def _upd(part, yT, dinvT, b, WnT):
    dout = WnT.shape[0]
    return pl.pallas_call(
        _upd_body,
        out_shape=(jax.ShapeDtypeStruct((32, N), jnp.float32),
                   jax.ShapeDtypeStruct((dout, NPS), jnp.float32)))(
        part, yT, dinvT, b.reshape(-1, 1), WnT)


def _upd4_body(p4_ref, y4T_ref, dinvT_ref, b4_ref, x1T_ref, x2T_ref, x3T_ref,
               batch_ref, xcT_ref, keyrow_ref, starts_ref, ends_ref):
    p = p4_ref[...]
    s4T = p[0:1, :N] + p[1:2, :N] + y4T_ref[0:1, :N]
    x4T = jnp.tanh(dinvT_ref[...] * s4T + b4_ref[...])
    bT = batch_ref[...]
    keyrow = bT.astype(jnp.float32) * 1e4 - x4T
    keyrow_ref[...] = jnp.concatenate(
        [keyrow, jnp.full((1, NPD - N), jnp.inf, jnp.float32)], axis=1)
    xcT = jnp.concatenate([x1T_ref[...], x2T_ref[...], x3T_ref[...], x4T,
                           jnp.zeros((31, N), jnp.float32)], axis=0)
    xcT_ref[...] = jnp.concatenate(
        [xcT, jnp.zeros((128, NPS - N), jnp.float32)], axis=1)
    g_iota = lax.broadcasted_iota(jnp.int32, (B, 1), 0)
    starts_ref[...] = jnp.sum((bT < g_iota).astype(jnp.float32),
                              axis=1, keepdims=True).reshape(1, B)
    ends_ref[...] = jnp.sum((bT <= g_iota).astype(jnp.float32),
                            axis=1, keepdims=True).reshape(1, B)


def _upd4(p4, y4T, dinvT, b4, x1T, x2T, x3T, batchT):
    return pl.pallas_call(
        _upd4_body,
        out_shape=(jax.ShapeDtypeStruct((128, NPS), jnp.float32),
                   jax.ShapeDtypeStruct((1, NPD), jnp.float32),
                   jax.ShapeDtypeStruct((1, B), jnp.float32),
                   jax.ShapeDtypeStruct((1, B), jnp.float32)))(
        p4, y4T, dinvT, b4.reshape(1, 1), x1T, x2T, x3T, batchT)


RCJ = 512


def _rank_body(ki_ref, kj_ref, bj_ref, bat_ref, starts_ref, ends_ref,
               pos_ref):
    i0 = pl.program_id(0) * 128
    ki = ki_ref[...]
    bat = bat_ref[...]
    iidx = (i0 + lax.broadcasted_iota(jnp.int32, (128, 1), 0)
            ).astype(jnp.float32)
    g_iota = lax.broadcasted_iota(jnp.int32, (1, B), 1)
    lo_f = jnp.sum(jnp.where(g_iota == bat[0:1, :], starts_ref[...], 0.0))
    hi_f = jnp.sum(jnp.where(g_iota == bat[127:128, :], ends_ref[...], 0.0))
    c_lo = lo_f.astype(jnp.int32) // RCJ
    c_hi = (hi_f.astype(jnp.int32) + (RCJ - 1)) // RCJ

    def chunk(c, acc):
        kj = kj_ref[pl.ds(c, 1), :]
        bj = bj_ref[pl.ds(c, 1), :]
        jidx = ((c * RCJ + lax.broadcasted_iota(jnp.int32, (1, RCJ), 1))
                ).astype(jnp.float32)
        t = (bj == bat) & ((kj < ki) | ((kj == ki) & (jidx < iidx)))
        return acc + jnp.sum(jnp.where(t, 1.0, 0.0), axis=1, keepdims=True)
    slot = lax.fori_loop(c_lo, c_hi, chunk, jnp.zeros((128, 1), jnp.float32))
    pos = jnp.where(slot < float(K), bat.astype(jnp.float32) * K + slot,
                    float(K * B))
    pos_ref[...] = pos.astype(jnp.int32)


def _rank(keyp, kj2, bj2, batchp, starts, ends):
    return pl.pallas_call(
        _rank_body,
        grid=(NPD // 128,),
        in_specs=[pl.BlockSpec((128, 1), lambda i: (i, 0)),
                  pl.BlockSpec((NPD // RCJ, RCJ), lambda i: (0, 0)),
                  pl.BlockSpec((NPD // RCJ, RCJ), lambda i: (0, 0)),
                  pl.BlockSpec((128, 1), lambda i: (i, 0)),
                  pl.BlockSpec((1, B), lambda i: (0, 0)),
                  pl.BlockSpec((1, B), lambda i: (0, 0))],
        out_specs=pl.BlockSpec((128, 1), lambda i: (i, 0)),
        out_shape=jax.ShapeDtypeStruct((NPD, 1), jnp.int32))(
        keyp, kj2, bj2, batchp, starts, ends)


def _head_kernel(g_ref, w5_ref, b5_ref, w6_ref, b6_ref, fc1_ref, fc1b_ref,
                 fc2_ref, fc2b_ref, out_ref):
    g = g_ref[...]
    h1 = jnp.maximum(g @ w5_ref[...] + b5_ref[...], 0.0)
    h2 = jnp.maximum(h1[:, :240], h1[:, 240:])
    h3 = jnp.maximum(h2 @ w6_ref[...] + b6_ref[...], 0.0)
    h4 = jnp.maximum(h3 @ fc1_ref[...] + fc1b_ref[...], 0.0)
    logits = h4 @ fc2_ref[...] + fc2b_ref[...]
    m = jnp.max(logits, axis=-1, keepdims=True)
    s = logits - m
    lse = jnp.log(jnp.sum(jnp.exp(s), axis=-1, keepdims=True))
    out_ref[...] = s - lse


def _head(g, conv5_w, conv5_b, conv6_w, conv6_b, fc1_w, fc1_b, fc2_w, fc2_b):
    w5 = conv5_w[:, 0, :].T
    w5big = jnp.zeros((K * 128, 480), jnp.float32)
    for t in range(K):
        cb = (t // 2) * 16 + (240 if t % 2 else 0)
        w5big = w5big.at[t * 128:t * 128 + 97, cb:cb + 16].set(w5)
    b5big = jnp.tile(conv5_b, K)
    w6big = jnp.zeros((240, 352), jnp.float32)
    for t in range(11):
        for i in range(5):
            blk = conv6_w[:, :, i].T
            w6big = w6big.at[(t + i) * 16:(t + i + 1) * 16,
                             t * 32:(t + 1) * 32].set(blk)
    b6big = jnp.tile(conv6_b, 11)
    fc1p = fc1_w.reshape(32, 11, 128).transpose(1, 0, 2).reshape(352, 128)
    return pl.pallas_call(
        _head_kernel,
        out_shape=jax.ShapeDtypeStruct((B, 10), jnp.float32),
    )(g, w5big, b5big, w6big, b6big, fc1p, fc1_b, fc2_w, fc2_b)


def kernel(x, edge_index, batch, W1, b1, W2, b2, W3, b3, W4, b4,
           conv5_w, conv5_b, conv6_w, conv6_b,
           fc1_w, fc1_b, fc2_w, fc2_b):
    pad = jnp.zeros((ERP * 128 - E,), jnp.int32)
    src2 = jnp.concatenate([edge_index[0], pad]).reshape(ERP, 128)
    dst2 = jnp.concatenate([edge_index[1], pad]).reshape(ERP, 128)
    deg = _sc_scalar(src2, dst2, None)
    xT = x.T
    y1T, dinvT = _pre(xT, W1.T, deg)
    p1 = _sc_rows(y1T.reshape(8, 4, NPS), src2, dst2)
    x1T, y2T = _upd(p1, y1T, dinvT, b1, W2.T)
    p2 = _sc_rows(y2T.reshape(8, 4, NPS), src2, dst2)
    x2T, y3T = _upd(p2, y2T, dinvT, b2, W3.T)
    p3 = _sc_rows(y3T.reshape(8, 4, NPS), src2, dst2)
    x3T, y4T = _upd(p3, y3T, dinvT, b3, W4.T)
    p4 = _sc_scalar(src2, dst2, y4T[0, :N])
    xcT, keyrow, starts, ends = _upd4(p4, y4T, dinvT, b4, x1T, x2T, x3T,
                                      batch.reshape(1, N))
    xc = xcT.T
    keyp = keyrow.T
    bpad = jnp.zeros((NPD - N,), jnp.int32)
    batchrow = jnp.concatenate([batch, bpad])
    pos = _rank(keyp, keyrow.reshape(NPD // RCJ, RCJ),
                batchrow.reshape(NPD // RCJ, RCJ),
                batchrow.reshape(NPD, 1), starts, ends)
    g = _table(pos.reshape(NPD // 128, 128), xc)
    return _head(g.reshape(B, K * 128), conv5_w, conv5_b, conv6_w, conv6_b,
                 fc1_w, fc1_b, fc2_w, fc2_b)
